# Initial kernel scaffold; baseline (speedup 1.0000x reference)
import functools

import jax
import jax.numpy as jnp
from jax import lax
from jax.experimental import pallas as pl
from jax.experimental.pallas import tpu as pltpu
from jax.experimental.pallas import tpu_sc as plsc

N = 10000
E = 320000
D = 128
H = 64

NC = 2
NS = 16
NW = NC * NS
PERW = E // NW
CHUNK = 80
NCH = PERW // CHUNK
NPAD = 10240
RPT = NPAD // NS

_relu = jax.nn.relu


def _dot(a, b):
    return jnp.dot(a, b, preferred_element_type=jnp.float32)


def _prep_body(nf, ga, Wv, bv, Wu, bu, Ae, Be, De, b1e, Dn, b1n,
               v_o, ta_o, tb_o, u_o, cue_o, cun_o):
    v = _relu(_dot(nf[...], Wv[...]) + bv[...])
    v_o[...] = v
    ta_o[...] = _dot(v, Ae[...])
    tb_o[...] = _dot(v, Be[...])
    u = _relu(_dot(ga[...], Wu[...]) + bu[...])
    u_o[...] = u
    cue_o[...] = _dot(u, De[...]) + b1e[...]
    cun_o[...] = _dot(u, Dn[...]) + b1n[...]


def _prep_call(nf, ga, Wv, bv, Wu, bu, Ae, Be, De, b1e, Dn, b1n):
    f32 = jnp.float32
    return pl.pallas_call(
        _prep_body,
        out_shape=[
            jax.ShapeDtypeStruct((N, H), f32),
            jax.ShapeDtypeStruct((N, H), f32),
            jax.ShapeDtypeStruct((N, H), f32),
            jax.ShapeDtypeStruct((1, H), f32),
            jax.ShapeDtypeStruct((1, H), f32),
            jax.ShapeDtypeStruct((1, H), f32),
        ],
    )(nf, ga, Wv, bv, Wu, bu, Ae, Be, De, b1e, Dn, b1n)


def _gather_body(ta_hbm, tb_hbm, src_hbm, dst_hbm, ga_hbm, gb_hbm,
                 idx_a, idx_b, buf_a, buf_b, sem_a, sem_b):
    c = lax.axis_index("c")
    s = lax.axis_index("s")
    w = s * NC + c
    base = w * PERW
    pltpu.sync_copy(src_hbm.at[w], idx_a)
    pltpu.sync_copy(dst_hbm.at[w], idx_b)

    def body(j, carry):
        cp_a = pltpu.async_copy(ta_hbm.at[idx_a.at[j]], buf_a, sem_a)
        cp_b = pltpu.async_copy(tb_hbm.at[idx_b.at[j]], buf_b, sem_b)
        cp_a.wait()
        cp_b.wait()
        pltpu.sync_copy(buf_a, ga_hbm.at[pl.ds(base + j * CHUNK, CHUNK)])
        pltpu.sync_copy(buf_b, gb_hbm.at[pl.ds(base + j * CHUNK, CHUNK)])
        return carry

    lax.fori_loop(0, NCH, body, 0)


def _gather_call(ta, tb, src3, dst3):
    f32 = jnp.float32
    mesh = plsc.VectorSubcoreMesh(core_axis_name="c", subcore_axis_name="s")
    fn = pl.kernel(
        _gather_body,
        out_type=[
            jax.ShapeDtypeStruct((E, H), f32),
            jax.ShapeDtypeStruct((E, H), f32),
        ],
        mesh=mesh,
        scratch_types=[
            pltpu.VMEM((NCH, CHUNK), jnp.int32),
            pltpu.VMEM((NCH, CHUNK), jnp.int32),
            pltpu.VMEM((CHUNK, H), f32),
            pltpu.VMEM((CHUNK, H), f32),
            pltpu.SemaphoreType.DMA,
            pltpu.SemaphoreType.DMA,
        ],
    )
    return fn(ta, tb, src3, dst3)


def _edge_body(ef, gA, gB, cue, We, be, Ce, W2e, b2e, Cn,
               oe, m2, ue):
    i = pl.program_id(0)
    e1 = _relu(_dot(ef[...], We[...]) + be[...])
    h1 = _relu(_dot(e1, Ce[...]) + gA[...] + gB[...] + cue[...])
    mij = _relu(_dot(h1, W2e[...]) + b2e[...])
    oe[...] = mij + ef[...]
    m2[...] = _dot(mij, Cn[...])

    @pl.when(i == 0)
    def _():
        ue[...] = jnp.zeros_like(ue)

    ue[...] += jnp.sum(mij, axis=0, keepdims=True)


def _edge_call(ef, gA, gB, cue, We, be, Ce, W2e, b2e, Cn, block_e=3200):
    f32 = jnp.float32
    grid = E // block_e
    full = lambda i: (0, 0)
    return pl.pallas_call(
        _edge_body,
        grid=(grid,),
        in_specs=[
            pl.BlockSpec((block_e, D), lambda i: (i, 0)),
            pl.BlockSpec((block_e, H), lambda i: (i, 0)),
            pl.BlockSpec((block_e, H), lambda i: (i, 0)),
            pl.BlockSpec((1, H), full),
            pl.BlockSpec((D, H), full),
            pl.BlockSpec((1, H), full),
            pl.BlockSpec((H, H), full),
            pl.BlockSpec((H, D), full),
            pl.BlockSpec((1, D), full),
            pl.BlockSpec((D, H), full),
        ],
        out_specs=[
            pl.BlockSpec((block_e, D), lambda i: (i, 0)),
            pl.BlockSpec((block_e, H), lambda i: (i, 0)),
            pl.BlockSpec((1, D), full),
        ],
        out_shape=[
            jax.ShapeDtypeStruct((E, D), f32),
            jax.ShapeDtypeStruct((E, H), f32),
            jax.ShapeDtypeStruct((1, D), f32),
        ],
    )(ef, gA, gB, cue, We, be, Ce, W2e, b2e, Cn)


def _scatter_body(m2_hbm, dst_hbm, zrow_hbm, z16_hbm, ones_hbm,
                  s_hbm, dg_hbm,
                  idx, mbuf, obuf, dbuf, ones, acc, dacc):
    c = lax.axis_index("c")
    s = lax.axis_index("s")
    w = s * NC + c
    base = w * PERW

    pltpu.sync_copy(zrow_hbm, obuf)
    pltpu.sync_copy(obuf, acc.at[pl.ds(s * RPT, RPT)])
    pltpu.sync_copy(z16_hbm, dbuf)
    pltpu.sync_copy(dbuf, dacc.at[pl.ds(s * RPT, RPT)])
    pltpu.sync_copy(ones_hbm, ones)
    pltpu.sync_copy(dst_hbm.at[w], idx)
    plsc.subcore_barrier()

    def body(j, carry):
        pltpu.sync_copy(m2_hbm.at[pl.ds(base + j * CHUNK, CHUNK)], mbuf)
        pltpu.sync_copy(mbuf, acc.at[idx.at[j]], add=True)
        pltpu.sync_copy(ones, dacc.at[idx.at[j]], add=True)
        return carry

    lax.fori_loop(0, NCH, body, 0)
    plsc.subcore_barrier()

    pltpu.sync_copy(acc.at[pl.ds(s * RPT, RPT)], obuf)
    pltpu.sync_copy(obuf, s_hbm.at[c].at[pl.ds(s * RPT, RPT)])
    pltpu.sync_copy(dacc.at[pl.ds(s * RPT, RPT)], dbuf)
    pltpu.sync_copy(dbuf, dg_hbm.at[c].at[pl.ds(s * RPT, RPT)])


def _scatter_call(m2, dst3, zrow, z16, ones):
    f32 = jnp.float32
    mesh = plsc.VectorSubcoreMesh(core_axis_name="c", subcore_axis_name="s")
    fn = pl.kernel(
        _scatter_body,
        out_type=[
            jax.ShapeDtypeStruct((NC, NPAD, H), f32),
            jax.ShapeDtypeStruct((NC, NPAD, 16), f32),
        ],
        mesh=mesh,
        scratch_types=[
            pltpu.VMEM((NCH, CHUNK), jnp.int32),
            pltpu.VMEM((CHUNK, H), f32),
            pltpu.VMEM((RPT, H), f32),
            pltpu.VMEM((RPT, 16), f32),
            pltpu.VMEM((CHUNK, 16), f32),
            pltpu.VMEM_SHARED((NPAD, H), f32),
            pltpu.VMEM_SHARED((NPAD, 16), f32),
        ],
    )
    return fn(m2, dst3, zrow, z16, ones)


def _node_body(nf, S, DG, u, ue_sum, ga, Wv, bv, An, cun, W2n, b2n,
               Wau, Wae, Wav, b1a, W2a, b2a, ov, ou):
    v = _relu(_dot(nf[...], Wv[...]) + bv[...])
    ssum = S[0, :N, :] + S[1, :N, :]
    deg = DG[0, :N, 0:1] + DG[1, :N, 0:1]
    ve_c = ssum / jnp.maximum(deg, 1.0)
    h = _relu(_dot(v, An[...]) + ve_c + cun[...])
    v_new = _relu(_dot(h, W2n[...]) + b2n[...])
    ov[...] = v_new + nf[...]
    uv = jnp.sum(v_new, axis=0, keepdims=True) * (1.0 / N)
    ue = ue_sum[...] * (1.0 / E)
    ha = _relu(_dot(u[...], Wau[...]) + _dot(ue, Wae[...])
               + _dot(uv, Wav[...]) + b1a[...])
    u_new = _relu(_dot(ha, W2a[...]) + b2a[...])
    ou[...] = u_new + ga[...]


def _node_call(nf, S, DG, u, ue_sum, ga, Wv, bv, An, cun, W2n, b2n,
               Wau, Wae, Wav, b1a, W2a, b2a):
    f32 = jnp.float32
    return pl.pallas_call(
        _node_body,
        out_shape=[
            jax.ShapeDtypeStruct((N, D), f32),
            jax.ShapeDtypeStruct((1, D), f32),
        ],
    )(nf, S, DG, u, ue_sum, ga, Wv, bv, An, cun, W2n, b2n,
      Wau, Wae, Wav, b1a, W2a, b2a)


def kernel(edge_feat, node_feat, graph_attr, params, edge_index):
    f32 = jnp.float32
    (We, be), = params["edge_dense"]
    (Wv, bv), = params["node_dense"]
    (Wu, bu), = params["attr_dense"]
    (W1e, b1e), (W2e, b2e) = params["conv_edge"]
    (W1n, b1n), (W2n, b2n) = params["conv_node"]
    (W1a, b1a), (W2a, b2a) = params["conv_attr"]

    Ae, Be, Ce, De = W1e[0:64], W1e[64:128], W1e[128:192], W1e[192:256]
    An, Cn, Dn = W1n[0:64], W1n[64:192], W1n[192:256]
    Wau, Wae, Wav = W1a[0:64], W1a[64:192], W1a[192:320]
    row = lambda b: b.reshape(1, -1)

    src3 = edge_index[0].reshape(NW, NCH, CHUNK)
    dst3 = edge_index[1].reshape(NW, NCH, CHUNK)

    v, ta, tb, u, cue, cun = _prep_call(
        node_feat, graph_attr, Wv, row(bv), Wu, row(bu),
        Ae, Be, De, row(b1e), Dn, row(b1n))

    gA, gB = _gather_call(ta, tb, src3, dst3)

    out_e, m2, ue_sum = _edge_call(
        edge_feat, gA, gB, cue, We, row(be), Ce, W2e, row(b2e), Cn)

    zrow = jnp.zeros((RPT, H), f32)
    z16 = jnp.zeros((RPT, 16), f32)
    ones = jnp.ones((CHUNK, 16), f32)
    S, DG = _scatter_call(m2, dst3, zrow, z16, ones)

    out_v, out_u = _node_call(
        node_feat, S, DG, u, ue_sum, graph_attr, Wv, row(bv), An, cun,
        W2n, row(b2n), Wau, Wae, Wav, row(b1a), W2a, row(b2a))

    return (out_e, out_v, out_u)


# speedup vs baseline: 2.6756x; 2.6756x over previous
import functools

import jax
import jax.numpy as jnp
from jax import lax
from jax.experimental import pallas as pl
from jax.experimental.pallas import tpu as pltpu
from jax.experimental.pallas import tpu_sc as plsc

N = 10000
E = 320000
D = 128
H = 64

NC = 2
NS = 16
NW = NC * NS
PERW = E // NW
CHUNK = 80
NCH = PERW // CHUNK
NPAD = 10240
RPT = NPAD // NS

_relu = jax.nn.relu


def _dot(a, b):
    return jnp.dot(a, b, preferred_element_type=jnp.float32)


def _prep_body(nf, ga, Wv, bv, Wu, bu, Ae, Be, De, b1e, Dn, b1n,
               v_o, ta_o, tb_o, u_o, cue_o, cun_o):
    v = _relu(_dot(nf[...], Wv[...]) + bv[...])
    v_o[...] = v
    ta_o[...] = _dot(v, Ae[...])
    tb_o[...] = _dot(v, Be[...])
    u = _relu(_dot(ga[...], Wu[...]) + bu[...])
    u_o[...] = u
    cue_o[...] = _dot(u, De[...]) + b1e[...]
    cun_o[...] = _dot(u, Dn[...]) + b1n[...]


def _prep_call(nf, ga, Wv, bv, Wu, bu, Ae, Be, De, b1e, Dn, b1n):
    f32 = jnp.float32
    return pl.pallas_call(
        _prep_body,
        out_shape=[
            jax.ShapeDtypeStruct((N, H), f32),
            jax.ShapeDtypeStruct((N, H), f32),
            jax.ShapeDtypeStruct((N, H), f32),
            jax.ShapeDtypeStruct((1, H), f32),
            jax.ShapeDtypeStruct((1, H), f32),
            jax.ShapeDtypeStruct((1, H), f32),
        ],
    )(nf, ga, Wv, bv, Wu, bu, Ae, Be, De, b1e, Dn, b1n)


def _gather_body(ta_hbm, tb_hbm, src_hbm, dst_hbm, ga_hbm, gb_hbm,
                 idx_a, idx_b, buf_a, buf_b, sem_a, sem_b):
    c = lax.axis_index("c")
    s = lax.axis_index("s")
    w = s * NC + c
    base = w * PERW
    pltpu.sync_copy(src_hbm.at[w], idx_a)
    pltpu.sync_copy(dst_hbm.at[w], idx_b)

    def body(j, carry):
        cp_a = pltpu.async_copy(ta_hbm.at[idx_a.at[j]], buf_a, sem_a)
        cp_b = pltpu.async_copy(tb_hbm.at[idx_b.at[j]], buf_b, sem_b)
        cp_a.wait()
        cp_b.wait()
        pltpu.sync_copy(buf_a, ga_hbm.at[pl.ds(base + j * CHUNK, CHUNK)])
        pltpu.sync_copy(buf_b, gb_hbm.at[pl.ds(base + j * CHUNK, CHUNK)])
        return carry

    lax.fori_loop(0, NCH, body, 0)


def _gather_call(ta, tb, src3, dst3):
    f32 = jnp.float32
    mesh = plsc.VectorSubcoreMesh(core_axis_name="c", subcore_axis_name="s")
    fn = pl.kernel(
        _gather_body,
        compiler_params=pltpu.CompilerParams(use_tc_tiling_on_sc=False),
        out_type=[
            jax.ShapeDtypeStruct((E, H), f32),
            jax.ShapeDtypeStruct((E, H), f32),
        ],
        mesh=mesh,
        scratch_types=[
            pltpu.VMEM((NCH, CHUNK), jnp.int32),
            pltpu.VMEM((NCH, CHUNK), jnp.int32),
            pltpu.VMEM((CHUNK, H), f32),
            pltpu.VMEM((CHUNK, H), f32),
            pltpu.SemaphoreType.DMA,
            pltpu.SemaphoreType.DMA,
        ],
    )
    return fn(ta, tb, src3, dst3)


def _edge_body(ef, gA, gB, cue, We, be, Ce, W2e, b2e, Cn,
               oe, m2, ue):
    i = pl.program_id(0)
    e1 = _relu(_dot(ef[...], We[...]) + be[...])
    h1 = _relu(_dot(e1, Ce[...]) + gA[...] + gB[...] + cue[...])
    mij = _relu(_dot(h1, W2e[...]) + b2e[...])
    oe[...] = mij + ef[...]
    m2[...] = _dot(mij, Cn[...])

    @pl.when(i == 0)
    def _():
        ue[...] = jnp.zeros_like(ue)

    ue[...] += jnp.sum(mij, axis=0, keepdims=True)


def _edge_call(ef, gA, gB, cue, We, be, Ce, W2e, b2e, Cn, block_e=3200):
    f32 = jnp.float32
    grid = E // block_e
    full = lambda i: (0, 0)
    return pl.pallas_call(
        _edge_body,
        grid=(grid,),
        in_specs=[
            pl.BlockSpec((block_e, D), lambda i: (i, 0)),
            pl.BlockSpec((block_e, H), lambda i: (i, 0)),
            pl.BlockSpec((block_e, H), lambda i: (i, 0)),
            pl.BlockSpec((1, H), full),
            pl.BlockSpec((D, H), full),
            pl.BlockSpec((1, H), full),
            pl.BlockSpec((H, H), full),
            pl.BlockSpec((H, D), full),
            pl.BlockSpec((1, D), full),
            pl.BlockSpec((D, H), full),
        ],
        out_specs=[
            pl.BlockSpec((block_e, D), lambda i: (i, 0)),
            pl.BlockSpec((block_e, H), lambda i: (i, 0)),
            pl.BlockSpec((1, D), full),
        ],
        out_shape=[
            jax.ShapeDtypeStruct((E, D), f32),
            jax.ShapeDtypeStruct((E, H), f32),
            jax.ShapeDtypeStruct((1, D), f32),
        ],
    )(ef, gA, gB, cue, We, be, Ce, W2e, b2e, Cn)


def _scatter_body(m2_hbm, dst_hbm, zrow_hbm, z16_hbm, ones_hbm,
                  s_hbm, dg_hbm,
                  idx, mbuf, obuf, dbuf, ones, acc, dacc):
    c = lax.axis_index("c")
    s = lax.axis_index("s")
    w = s * NC + c
    base = w * PERW

    pltpu.sync_copy(zrow_hbm, obuf)
    pltpu.sync_copy(obuf, acc.at[pl.ds(s * RPT, RPT)])
    pltpu.sync_copy(z16_hbm, dbuf)
    pltpu.sync_copy(dbuf, dacc.at[pl.ds(s * RPT, RPT)])
    pltpu.sync_copy(ones_hbm, ones)
    pltpu.sync_copy(dst_hbm.at[w], idx)
    plsc.subcore_barrier()

    def body(j, carry):
        pltpu.sync_copy(m2_hbm.at[pl.ds(base + j * CHUNK, CHUNK)], mbuf)
        pltpu.sync_copy(mbuf, acc.at[idx.at[j]], add=True)
        pltpu.sync_copy(ones, dacc.at[idx.at[j]], add=True)
        return carry

    lax.fori_loop(0, NCH, body, 0)
    plsc.subcore_barrier()

    pltpu.sync_copy(acc.at[pl.ds(s * RPT, RPT)], obuf)
    pltpu.sync_copy(obuf, s_hbm.at[c].at[pl.ds(s * RPT, RPT)])
    pltpu.sync_copy(dacc.at[pl.ds(s * RPT, RPT)], dbuf)
    pltpu.sync_copy(dbuf, dg_hbm.at[c].at[pl.ds(s * RPT, RPT)])


def _scatter_call(m2, dst3, zrow, z16, ones):
    f32 = jnp.float32
    mesh = plsc.VectorSubcoreMesh(core_axis_name="c", subcore_axis_name="s")
    fn = pl.kernel(
        _scatter_body,
        compiler_params=pltpu.CompilerParams(use_tc_tiling_on_sc=False),
        out_type=[
            jax.ShapeDtypeStruct((NC, NPAD, H), f32),
            jax.ShapeDtypeStruct((NC, NPAD, 16), f32),
        ],
        mesh=mesh,
        scratch_types=[
            pltpu.VMEM((NCH, CHUNK), jnp.int32),
            pltpu.VMEM((CHUNK, H), f32),
            pltpu.VMEM((RPT, H), f32),
            pltpu.VMEM((RPT, 16), f32),
            pltpu.VMEM((CHUNK, 16), f32),
            pltpu.VMEM_SHARED((NPAD, H), f32),
            pltpu.VMEM_SHARED((NPAD, 16), f32),
        ],
    )
    return fn(m2, dst3, zrow, z16, ones)


def _node_body(nf, S, DG, u, ue_sum, ga, Wv, bv, An, cun, W2n, b2n,
               Wau, Wae, Wav, b1a, W2a, b2a, ov, ou):
    v = _relu(_dot(nf[...], Wv[...]) + bv[...])
    ssum = S[0, :N, :] + S[1, :N, :]
    deg = DG[0, :N, 0:1] + DG[1, :N, 0:1]
    ve_c = ssum / jnp.maximum(deg, 1.0)
    h = _relu(_dot(v, An[...]) + ve_c + cun[...])
    v_new = _relu(_dot(h, W2n[...]) + b2n[...])
    ov[...] = v_new + nf[...]
    uv = jnp.sum(v_new, axis=0, keepdims=True) * (1.0 / N)
    ue = ue_sum[...] * (1.0 / E)
    ha = _relu(_dot(u[...], Wau[...]) + _dot(ue, Wae[...])
               + _dot(uv, Wav[...]) + b1a[...])
    u_new = _relu(_dot(ha, W2a[...]) + b2a[...])
    ou[...] = u_new + ga[...]


def _node_call(nf, S, DG, u, ue_sum, ga, Wv, bv, An, cun, W2n, b2n,
               Wau, Wae, Wav, b1a, W2a, b2a):
    f32 = jnp.float32
    return pl.pallas_call(
        _node_body,
        out_shape=[
            jax.ShapeDtypeStruct((N, D), f32),
            jax.ShapeDtypeStruct((1, D), f32),
        ],
    )(nf, S, DG, u, ue_sum, ga, Wv, bv, An, cun, W2n, b2n,
      Wau, Wae, Wav, b1a, W2a, b2a)


def kernel(edge_feat, node_feat, graph_attr, params, edge_index):
    f32 = jnp.float32
    (We, be), = params["edge_dense"]
    (Wv, bv), = params["node_dense"]
    (Wu, bu), = params["attr_dense"]
    (W1e, b1e), (W2e, b2e) = params["conv_edge"]
    (W1n, b1n), (W2n, b2n) = params["conv_node"]
    (W1a, b1a), (W2a, b2a) = params["conv_attr"]

    Ae, Be, Ce, De = W1e[0:64], W1e[64:128], W1e[128:192], W1e[192:256]
    An, Cn, Dn = W1n[0:64], W1n[64:192], W1n[192:256]
    Wau, Wae, Wav = W1a[0:64], W1a[64:192], W1a[192:320]
    row = lambda b: b.reshape(1, -1)

    src3 = edge_index[0].reshape(NW, NCH, CHUNK)
    dst3 = edge_index[1].reshape(NW, NCH, CHUNK)

    v, ta, tb, u, cue, cun = _prep_call(
        node_feat, graph_attr, Wv, row(bv), Wu, row(bu),
        Ae, Be, De, row(b1e), Dn, row(b1n))

    gA, gB = _gather_call(ta, tb, src3, dst3)

    out_e, m2, ue_sum = _edge_call(
        edge_feat, gA, gB, cue, We, row(be), Ce, W2e, row(b2e), Cn)

    zrow = jnp.zeros((RPT, H), f32)
    z16 = jnp.zeros((RPT, 16), f32)
    ones = jnp.ones((CHUNK, 16), f32)
    S, DG = _scatter_call(m2, dst3, zrow, z16, ones)

    out_v, out_u = _node_call(
        node_feat, S, DG, u, ue_sum, graph_attr, Wv, row(bv), An, cun,
        W2n, row(b2n), Wau, Wae, Wav, row(b1a), W2a, row(b2a))

    return (out_e, out_v, out_u)


# speedup vs baseline: 4.0821x; 1.5257x over previous
import functools

import jax
import jax.numpy as jnp
from jax import lax
from jax.experimental import pallas as pl
from jax.experimental.pallas import tpu as pltpu
from jax.experimental.pallas import tpu_sc as plsc

N = 10000
E = 320000
D = 128
H = 64

NC = 2
NS = 16
NW = NC * NS
PERW = E // NW

GCH = 50
GIDX = 2 * GCH
GNCH = PERW // GCH
GSLOTS = 5
GITER = GNCH // GSLOTS

SCH = 80
SNCH = PERW // SCH
SSLOTS = 5
SITER = SNCH // SSLOTS

BLOCK_E = 3200
EPAIR = BLOCK_E // 2

NPAD = 10240
RPT = NPAD // NS

_relu = jax.nn.relu


def _dot(a, b):
    return jnp.dot(a, b, preferred_element_type=jnp.float32)


def _prep_body(nf, ga, Wv, bv, Wu, bu, ABe, De, b1e, Dn, b1n,
               t_o, u_o, cue_o, cun_o):
    v = _relu(_dot(nf[...], Wv[...]) + bv[...])
    t_o[...] = _dot(v, ABe[...])
    u = _relu(_dot(ga[...], Wu[...]) + bu[...])
    u_o[...] = u
    cue_o[...] = _dot(u, De[...]) + b1e[...]
    cun_o[...] = _dot(u, Dn[...]) + b1n[...]


def _prep_call(nf, ga, Wv, bv, Wu, bu, ABe, De, b1e, Dn, b1n):
    f32 = jnp.float32
    return pl.pallas_call(
        _prep_body,
        out_shape=[
            jax.ShapeDtypeStruct((N, D), f32),
            jax.ShapeDtypeStruct((1, H), f32),
            jax.ShapeDtypeStruct((1, H), f32),
            jax.ShapeDtypeStruct((1, H), f32),
        ],
    )(nf, ga, Wv, bv, Wu, bu, ABe, De, b1e, Dn, b1n)


def _gather_body(t2_hbm, idx_hbm, dst_hbm, z16_hbm, ones_hbm,
                 g2_hbm, dg_hbm,
                 idx, b0, b1, b2, b3, b4,
                 idxd, dbuf, ones, dsem,
                 g0, g1, g2s, g3, g4, w0, w1, w2, w3, w4, dacc):
    c = lax.axis_index("c")
    s = lax.axis_index("s")
    w = s * NC + c
    obase = w * PERW * 2
    bufs = (b0, b1, b2, b3, b4)
    gsems = (g0, g1, g2s, g3, g4)
    wsems = (w0, w1, w2, w3, w4)
    pltpu.sync_copy(idx_hbm.at[w], idx)

    pltpu.sync_copy(z16_hbm, dbuf)
    pltpu.sync_copy(dbuf, dacc.at[pl.ds(s * RPT, RPT)])
    pltpu.sync_copy(ones_hbm, ones)
    pltpu.sync_copy(dst_hbm.at[w], idxd)
    plsc.subcore_barrier()

    def deg_round(r, carry):
        for t in range(25):
            pltpu.async_copy(ones, dacc.at[idxd.at[r * 25 + t]], dsem,
                             add=True)
        for t in range(25):
            pltpu.make_async_copy(ones, dacc.at[idxd.at[r * 25 + t]],
                                  dsem).wait()
        return carry

    lax.fori_loop(0, SNCH // 25, deg_round, 0)

    def g_start(ch, k):
        pltpu.async_copy(t2_hbm.at[idx.at[ch]], bufs[k], gsems[k])

    def g_wait(ch, k):
        pltpu.make_async_copy(t2_hbm.at[idx.at[ch]], bufs[k],
                              gsems[k]).wait()

    def w_rows(ch):
        return g2_hbm.at[pl.ds(obase + ch * GIDX, GIDX)]

    for k in range(GSLOTS):
        g_start(k, k)

    def body(i, carry):
        ch0 = i * GSLOTS
        for k in range(GSLOTS):
            g_wait(ch0 + k, k)
            pltpu.async_copy(bufs[k], w_rows(ch0 + k), wsems[k])
        for k in range(GSLOTS):
            @pl.when(i < GITER - 1)
            def _():
                pltpu.make_async_copy(bufs[k], w_rows(ch0 + k),
                                      wsems[k]).wait()
                g_start(ch0 + GSLOTS + k, k)
        return carry

    lax.fori_loop(0, GITER, body, 0)
    last = (GITER - 1) * GSLOTS
    for k in range(GSLOTS):
        pltpu.make_async_copy(bufs[k], w_rows(last + k), wsems[k]).wait()

    plsc.subcore_barrier()
    pltpu.sync_copy(dacc.at[pl.ds(s * RPT, RPT)], dbuf)
    pltpu.sync_copy(dbuf, dg_hbm.at[c].at[pl.ds(s * RPT, RPT)])


def _gather_call(t2, idx3, dst3, z16, ones):
    f32 = jnp.float32
    mesh = plsc.VectorSubcoreMesh(core_axis_name="c", subcore_axis_name="s")
    fn = pl.kernel(
        _gather_body,
        out_type=[
            jax.ShapeDtypeStruct((2 * E, H), f32),
            jax.ShapeDtypeStruct((NC, NPAD, 16), f32),
        ],
        mesh=mesh,
        compiler_params=pltpu.CompilerParams(use_tc_tiling_on_sc=False),
        scratch_types=(
            [pltpu.VMEM((GNCH, GIDX), jnp.int32)]
            + [pltpu.VMEM((GIDX, H), f32) for _ in range(GSLOTS)]
            + [pltpu.VMEM((SNCH, SCH), jnp.int32),
               pltpu.VMEM((RPT, 16), f32),
               pltpu.VMEM((SCH, 16), f32),
               pltpu.SemaphoreType.DMA]
            + [pltpu.SemaphoreType.DMA for _ in range(2 * GSLOTS)]
            + [pltpu.VMEM_SHARED((NPAD, 16), f32)]
        ),
    )
    return fn(t2, idx3, dst3, z16, ones)


def _edge_body(ef, g, cue, We, be, Ce, W2e, b2e, Cn,
               oe, m2, ue):
    i = pl.program_id(0)
    e1 = _relu(_dot(ef[...], We[...]) + be[...])
    gg = g[...]
    h1 = _relu(_dot(e1, Ce[...]) + gg[:, :H] + gg[:, H:] + cue[...])
    mij = _relu(_dot(h1, W2e[...]) + b2e[...])
    oe[...] = mij + ef[...]
    nb = mij.shape[0] // 2
    m2[...] = jnp.concatenate(
        [_dot(mij[:nb], Cn[...]), _dot(mij[nb:], Cn[...])], axis=1)

    @pl.when(i == 0)
    def _():
        ue[...] = jnp.zeros_like(ue)

    ue[...] += jnp.sum(mij, axis=0, keepdims=True)


def _edge_call(ef, g, cue, We, be, Ce, W2e, b2e, Cn, block_e=BLOCK_E):
    f32 = jnp.float32
    grid = E // block_e
    full = lambda i: (0, 0)
    return pl.pallas_call(
        _edge_body,
        grid=(grid,),
        in_specs=[
            pl.BlockSpec((block_e, D), lambda i: (i, 0)),
            pl.BlockSpec((block_e, D), lambda i: (i, 0)),
            pl.BlockSpec((1, H), full),
            pl.BlockSpec((D, H), full),
            pl.BlockSpec((1, H), full),
            pl.BlockSpec((H, H), full),
            pl.BlockSpec((H, D), full),
            pl.BlockSpec((1, D), full),
            pl.BlockSpec((D, H), full),
        ],
        out_specs=[
            pl.BlockSpec((block_e, D), lambda i: (i, 0)),
            pl.BlockSpec((block_e // 2, D), lambda i: (i, 0)),
            pl.BlockSpec((1, D), full),
        ],
        out_shape=[
            jax.ShapeDtypeStruct((E, D), f32),
            jax.ShapeDtypeStruct((E // 2, D), f32),
            jax.ShapeDtypeStruct((1, D), f32),
        ],
    )(ef, g, cue, We, be, Ce, W2e, b2e, Cn)


def _scatter_body(m2_hbm, dst_hbm, zrow_hbm,
                  s_hbm,
                  idx, m0, m1, m2b, m3, m4, obuf,
                  l0, l1, l2, l3, l4, s0, s1, s2, s3, s4,
                  acc):
    c = lax.axis_index("c")
    s = lax.axis_index("s")
    w = s * NC + c
    base = w * PERW
    mbufs = (m0, m1, m2b, m3, m4)
    lsems = (l0, l1, l2, l3, l4)
    ssems = (s0, s1, s2, s3, s4)

    pltpu.sync_copy(zrow_hbm, obuf)
    pltpu.sync_copy(obuf, acc.at[pl.ds(s * RPT, RPT)])
    pltpu.sync_copy(dst_hbm.at[w], idx)
    plsc.subcore_barrier()

    def rows(ch):
        return m2_hbm.at[pl.ds(base + ch * SCH, SCH)]

    def l_start(ch, k):
        pltpu.async_copy(rows(ch), mbufs[k], lsems[k])

    for k in range(SSLOTS):
        l_start(k, k)

    def body(i, carry):
        ch0 = i * SSLOTS
        for k in range(SSLOTS):
            pltpu.make_async_copy(rows(ch0 + k), mbufs[k], lsems[k]).wait()
            pltpu.async_copy(mbufs[k], acc.at[idx.at[ch0 + k]], ssems[k],
                             add=True)
        for k in range(SSLOTS):
            @pl.when(i < SITER - 1)
            def _():
                pltpu.make_async_copy(mbufs[k], acc.at[idx.at[ch0 + k]],
                                      ssems[k]).wait()
                l_start(ch0 + SSLOTS + k, k)
        return carry

    lax.fori_loop(0, SITER, body, 0)
    last = (SITER - 1) * SSLOTS
    for k in range(SSLOTS):
        pltpu.make_async_copy(mbufs[k], acc.at[idx.at[last + k]],
                              ssems[k]).wait()
    plsc.subcore_barrier()

    pltpu.sync_copy(acc.at[pl.ds(s * RPT, RPT)], obuf)
    pltpu.sync_copy(obuf, s_hbm.at[c].at[pl.ds(s * RPT, RPT)])


def _scatter_call(m2, dst3, zrow):
    f32 = jnp.float32
    mesh = plsc.VectorSubcoreMesh(core_axis_name="c", subcore_axis_name="s")
    fn = pl.kernel(
        _scatter_body,
        out_type=[
            jax.ShapeDtypeStruct((NC, NPAD, H), f32),
        ],
        mesh=mesh,
        compiler_params=pltpu.CompilerParams(use_tc_tiling_on_sc=False),
        scratch_types=(
            [pltpu.VMEM((SNCH, SCH), jnp.int32)]
            + [pltpu.VMEM((SCH, H), f32) for _ in range(SSLOTS)]
            + [pltpu.VMEM((RPT, H), f32)]
            + [pltpu.SemaphoreType.DMA for _ in range(2 * SSLOTS)]
            + [pltpu.VMEM_SHARED((NPAD, H), f32)]
        ),
    )
    return fn(m2, dst3, zrow)


def _node_body(nf, S, DG, u, ue_sum, ga, Wv, bv, An, cun, W2n, b2n,
               Wau, Wae, Wav, b1a, W2a, b2a, ov, ou):
    v = _relu(_dot(nf[...], Wv[...]) + bv[...])
    ssum = S[0, :N, :] + S[1, :N, :]
    deg = DG[0, :N, 0:1] + DG[1, :N, 0:1]
    ve_c = ssum / jnp.maximum(deg, 1.0)
    h = _relu(_dot(v, An[...]) + ve_c + cun[...])
    v_new = _relu(_dot(h, W2n[...]) + b2n[...])
    ov[...] = v_new + nf[...]
    uv = jnp.sum(v_new, axis=0, keepdims=True) * (1.0 / N)
    ue = ue_sum[...] * (1.0 / E)
    ha = _relu(_dot(u[...], Wau[...]) + _dot(ue, Wae[...])
               + _dot(uv, Wav[...]) + b1a[...])
    u_new = _relu(_dot(ha, W2a[...]) + b2a[...])
    ou[...] = u_new + ga[...]


def _node_call(nf, S, DG, u, ue_sum, ga, Wv, bv, An, cun, W2n, b2n,
               Wau, Wae, Wav, b1a, W2a, b2a):
    f32 = jnp.float32
    return pl.pallas_call(
        _node_body,
        out_shape=[
            jax.ShapeDtypeStruct((N, D), f32),
            jax.ShapeDtypeStruct((1, D), f32),
        ],
    )(nf, S, DG, u, ue_sum, ga, Wv, bv, An, cun, W2n, b2n,
      Wau, Wae, Wav, b1a, W2a, b2a)


def kernel(edge_feat, node_feat, graph_attr, params, edge_index):
    f32 = jnp.float32
    (We, be), = params["edge_dense"]
    (Wv, bv), = params["node_dense"]
    (Wu, bu), = params["attr_dense"]
    (W1e, b1e), (W2e, b2e) = params["conv_edge"]
    (W1n, b1n), (W2n, b2n) = params["conv_node"]
    (W1a, b1a), (W2a, b2a) = params["conv_attr"]

    ABe = jnp.concatenate([W1e[0:64], W1e[64:128]], axis=1)
    Ce, De = W1e[128:192], W1e[192:256]
    An, Cn, Dn = W1n[0:64], W1n[64:192], W1n[192:256]
    Wau, Wae, Wav = W1a[0:64], W1a[64:192], W1a[192:320]
    row = lambda b: b.reshape(1, -1)

    src = edge_index[0]
    dst = edge_index[1]
    idx3 = jnp.stack([src * 2, dst * 2 + 1], axis=1).reshape(NW, GNCH, GIDX)
    dst3 = dst.reshape(NW, SNCH, SCH)
    dstp3 = (dst.reshape(E // (2 * EPAIR), 2, EPAIR)
             .transpose(0, 2, 1).reshape(NW, SNCH, SCH))

    tcat, u, cue, cun = _prep_call(
        node_feat, graph_attr, Wv, row(bv), Wu, row(bu),
        ABe, De, row(b1e), Dn, row(b1n))
    t2 = tcat.reshape(2 * N, H)

    z16 = jnp.zeros((RPT, 16), f32)
    ones = jnp.ones((SCH, 16), f32)
    g2, DG = _gather_call(t2, idx3, dst3, z16, ones)
    gAB = g2.reshape(E, D)

    out_e, m2p, ue_sum = _edge_call(
        edge_feat, gAB, cue, We, row(be), Ce, W2e, row(b2e), Cn)
    m2 = m2p.reshape(E, H)

    zrow = jnp.zeros((RPT, H), f32)
    S, = _scatter_call(m2, dstp3, zrow)

    out_v, out_u = _node_call(
        node_feat, S, DG, u, ue_sum, graph_attr, Wv, row(bv), An, cun,
        W2n, row(b2n), Wau, Wae, Wav, row(b1a), W2a, row(b2a))

    return (out_e, out_v, out_u)


# speedup vs baseline: 6.2017x; 1.5192x over previous
import functools

import jax
import jax.numpy as jnp
from jax import lax
from jax.experimental import pallas as pl
from jax.experimental.pallas import tpu as pltpu
from jax.experimental.pallas import tpu_sc as plsc

N = 10000
E = 320000
EH = E // 2
D = 128
H = 64

NC = 2
NS = 16
NW = NC * NS

PERW_G = EH // NW
GCH = 40
GIDX = 2 * GCH
GNCH = PERW_G // GCH
GSLOTS = 5
GITER = GNCH // GSLOTS

PERW = 10000
SCH = 80
SNCH = PERW // SCH
SSLOTS = 5
SITER = SNCH // SSLOTS

BLOCK_E = 6400
GRID_H = EH // BLOCK_E
EPAIR = BLOCK_E // 2
DCOV = 19200
DPAD = 6400

NPAD = 10240
RPT = NPAD // NS

_relu = jax.nn.relu


def _dot(a, b):
    return jnp.dot(a, b, preferred_element_type=jnp.float32)


def _prep_body(nf, ga, Wv, bv, Wu, bu, ABe, De, b1e, Dn, b1n,
               t_o, u_o, cue_o, cun_o):
    v = _relu(_dot(nf[...], Wv[...]) + bv[...])
    t_o[...] = _dot(v, ABe[...])
    u = _relu(_dot(ga[...], Wu[...]) + bu[...])
    u_o[...] = u
    cue_o[...] = _dot(u, De[...]) + b1e[...]
    cun_o[...] = _dot(u, Dn[...]) + b1n[...]


def _prep_call(nf, ga, Wv, bv, Wu, bu, ABe, De, b1e, Dn, b1n):
    f32 = jnp.float32
    return pl.pallas_call(
        _prep_body,
        out_shape=[
            jax.ShapeDtypeStruct((N, D), f32),
            jax.ShapeDtypeStruct((1, H), f32),
            jax.ShapeDtypeStruct((1, H), f32),
            jax.ShapeDtypeStruct((1, H), f32),
        ],
    )(nf, ga, Wv, bv, Wu, bu, ABe, De, b1e, Dn, b1n)


def _gather_core(t2_hbm, g2_hbm, srcv, dstv, idxv, bufs, gsems, wsems,
                 base, obase):
    def bld(ch, carry):
        for g in range(GIDX // 16):
            p = lax.iota(jnp.int32, 16) + (ch * GIDX + 16 * g)
            e = lax.shift_right_logical(p, 1)
            par = lax.bitwise_and(p, 1)
            sv = plsc.load_gather(srcv, [e])
            dv = plsc.load_gather(dstv, [e])
            idxv[ch, pl.ds(16 * g, 16)] = jnp.where(par == 1,
                                                    2 * dv + 1, 2 * sv)
        return carry

    lax.fori_loop(0, GNCH, bld, 0)

    def g_start(ch, k):
        pltpu.async_copy(t2_hbm.at[idxv.at[ch]], bufs[k], gsems[k])

    def g_wait(ch, k):
        pltpu.make_async_copy(t2_hbm.at[idxv.at[ch]], bufs[k],
                              gsems[k]).wait()

    def w_rows(ch):
        return g2_hbm.at[pl.ds(obase + ch * GIDX, GIDX)]

    for k in range(GSLOTS):
        g_start(k, k)

    def body(i, carry):
        ch0 = i * GSLOTS
        for k in range(GSLOTS):
            g_wait(ch0 + k, k)
            pltpu.async_copy(bufs[k], w_rows(ch0 + k), wsems[k])
        for k in range(GSLOTS):
            @pl.when(i < GITER - 1)
            def _():
                pltpu.make_async_copy(bufs[k], w_rows(ch0 + k),
                                      wsems[k]).wait()
                g_start(ch0 + GSLOTS + k, k)
        return carry

    lax.fori_loop(0, GITER, body, 0)
    last = (GITER - 1) * GSLOTS
    for k in range(GSLOTS):
        pltpu.make_async_copy(bufs[k], w_rows(last + k), wsems[k]).wait()


def _gather_a_body(t2_hbm, src_hbm, dst_hbm, dstf_hbm, z16_hbm, ones_hbm,
                   g2_hbm, dg_hbm,
                   srcv, dstv, dstd, idxv, b0, b1, b2, b3, b4,
                   idxd, dbuf, ones, dsem,
                   g0, g1, g2s, g3, g4, w0, w1, w2, w3, w4, dacc):
    c = lax.axis_index("c")
    s = lax.axis_index("s")
    w = s * NC + c
    base = w * PERW_G
    obase = 2 * base
    bufs = (b0, b1, b2, b3, b4)
    gsems = (g0, g1, g2s, g3, g4)
    wsems = (w0, w1, w2, w3, w4)

    pltpu.sync_copy(src_hbm.at[pl.ds(base, PERW_G)], srcv)
    pltpu.sync_copy(dst_hbm.at[pl.ds(base, PERW_G)], dstv)

    pltpu.sync_copy(z16_hbm, dbuf)
    pltpu.sync_copy(dbuf, dacc.at[pl.ds(s * RPT, RPT)])
    pltpu.sync_copy(ones_hbm, ones)
    pltpu.sync_copy(dstf_hbm.at[pl.ds(w * PERW, PERW)], dstd)

    def bld_d(j, carry):
        for g in range(SCH // 16):
            idxd[j, pl.ds(16 * g, 16)] = dstd[pl.ds(j * SCH + 16 * g, 16)]
        return carry

    lax.fori_loop(0, SNCH, bld_d, 0)
    plsc.subcore_barrier()

    def deg_round(r, carry):
        for t in range(25):
            pltpu.async_copy(ones, dacc.at[idxd.at[r * 25 + t]], dsem,
                             add=True)
        for t in range(25):
            pltpu.make_async_copy(ones, dacc.at[idxd.at[r * 25 + t]],
                                  dsem).wait()
        return carry

    lax.fori_loop(0, SNCH // 25, deg_round, 0)

    _gather_core(t2_hbm, g2_hbm, srcv, dstv, idxv, bufs, gsems, wsems,
                 base, obase)

    plsc.subcore_barrier()
    pltpu.sync_copy(dacc.at[pl.ds(s * RPT, RPT)], dbuf)
    pltpu.sync_copy(dbuf, dg_hbm.at[c].at[pl.ds(s * RPT, RPT)])


def _gather_b_body(t2_hbm, src_hbm, dst_hbm,
                   g2_hbm,
                   srcv, dstv, idxv, b0, b1, b2, b3, b4,
                   g0, g1, g2s, g3, g4, w0, w1, w2, w3, w4):
    c = lax.axis_index("c")
    s = lax.axis_index("s")
    w = s * NC + c
    base = w * PERW_G
    obase = 2 * base
    bufs = (b0, b1, b2, b3, b4)
    gsems = (g0, g1, g2s, g3, g4)
    wsems = (w0, w1, w2, w3, w4)
    pltpu.sync_copy(src_hbm.at[pl.ds(base, PERW_G)], srcv)
    pltpu.sync_copy(dst_hbm.at[pl.ds(base, PERW_G)], dstv)
    _gather_core(t2_hbm, g2_hbm, srcv, dstv, idxv, bufs, gsems, wsems,
                 base, obase)


_SC_PARAMS = pltpu.CompilerParams(use_tc_tiling_on_sc=False,
                                  needs_layout_passes=False)
_MESH = plsc.VectorSubcoreMesh(core_axis_name="c", subcore_axis_name="s")


def _gather_a_call(t2, src_h, dst_h, dst_full, z16, ones):
    f32 = jnp.float32
    fn = pl.kernel(
        _gather_a_body,
        out_type=[
            jax.ShapeDtypeStruct((2 * EH, H), f32),
            jax.ShapeDtypeStruct((NC, NPAD, 16), f32),
        ],
        mesh=_MESH,
        compiler_params=_SC_PARAMS,
        scratch_types=(
            [pltpu.VMEM((PERW_G,), jnp.int32),
             pltpu.VMEM((PERW_G,), jnp.int32),
             pltpu.VMEM((PERW,), jnp.int32),
             pltpu.VMEM((GNCH, GIDX), jnp.int32)]
            + [pltpu.VMEM((GIDX, H), f32) for _ in range(GSLOTS)]
            + [pltpu.VMEM((SNCH, SCH), jnp.int32),
               pltpu.VMEM((RPT, 16), f32),
               pltpu.VMEM((SCH, 16), f32),
               pltpu.SemaphoreType.DMA]
            + [pltpu.SemaphoreType.DMA for _ in range(2 * GSLOTS)]
            + [pltpu.VMEM_SHARED((NPAD, 16), f32)]
        ),
    )
    return fn(t2, src_h, dst_h, dst_full, z16, ones)


def _gather_b_call(t2, src_h, dst_h):
    f32 = jnp.float32
    fn = pl.kernel(
        _gather_b_body,
        out_type=[jax.ShapeDtypeStruct((2 * EH, H), f32)],
        mesh=_MESH,
        compiler_params=_SC_PARAMS,
        scratch_types=(
            [pltpu.VMEM((PERW_G,), jnp.int32),
             pltpu.VMEM((PERW_G,), jnp.int32),
             pltpu.VMEM((GNCH, GIDX), jnp.int32)]
            + [pltpu.VMEM((GIDX, H), f32) for _ in range(GSLOTS)]
            + [pltpu.SemaphoreType.DMA for _ in range(2 * GSLOTS)]
        ),
    )
    return fn(t2, src_h, dst_h)


def _edge_body_a(ef, g, cue, We, be, Ce, W2e, b2e, Cn, oe, m2, ue):
    _edge_compute(ef, g, cue, We, be, Ce, W2e, b2e, Cn, oe, m2, ue)


def _edge_body_b(seed, ef, g, cue, We, be, Ce, W2e, b2e, Cn, oe, m2, ue):
    _edge_compute(ef, g, cue, We, be, Ce, W2e, b2e, Cn, oe, m2, ue)


def _edge_compute(ef, g, cue, We, be, Ce, W2e, b2e, Cn, oe, m2, ue):
    i = pl.program_id(0)
    e1 = _relu(_dot(ef[...], We[...]) + be[...])
    gg = g[...]
    h1 = _relu(_dot(e1, Ce[...]) + gg[:, :H] + gg[:, H:] + cue[...])
    mij = _relu(_dot(h1, W2e[...]) + b2e[...])
    oe[...] = mij + ef[...]
    nb = mij.shape[0] // 2
    m2[...] = jnp.concatenate(
        [_dot(mij[:nb], Cn[...]), _dot(mij[nb:], Cn[...])], axis=1)

    @pl.when(i == 0)
    def _():
        ue[...] = jnp.zeros_like(ue)

    ue[...] += jnp.sum(mij, axis=0, keepdims=True)


def _edge_call(ef, g, cue, We, be, Ce, W2e, b2e, Cn, half, seed=None):
    f32 = jnp.float32
    full = lambda i: (0, 0)
    off = half * GRID_H
    in_specs = [
        pl.BlockSpec((BLOCK_E, D), lambda i: (i + off, 0)),
        pl.BlockSpec((BLOCK_E, D), lambda i: (i, 0)),
        pl.BlockSpec((1, H), full),
        pl.BlockSpec((D, H), full),
        pl.BlockSpec((1, H), full),
        pl.BlockSpec((H, H), full),
        pl.BlockSpec((H, D), full),
        pl.BlockSpec((1, D), full),
        pl.BlockSpec((D, H), full),
    ]
    args = [ef, g, cue, We, be, Ce, W2e, b2e, Cn]
    kwargs = {}
    if seed is None:
        body = _edge_body_a
    else:
        body = _edge_body_b
        in_specs = [pl.BlockSpec((8, D), full)] + in_specs
        args = [seed] + args
        kwargs["input_output_aliases"] = {0: 0}
    return pl.pallas_call(
        body,
        grid=(GRID_H,),
        in_specs=in_specs,
        out_specs=[
            pl.BlockSpec((BLOCK_E, D), lambda i: (i + off, 0)),
            pl.BlockSpec((BLOCK_E // 2, D), lambda i: (i, 0)),
            pl.BlockSpec((1, D), full),
        ],
        out_shape=[
            jax.ShapeDtypeStruct((E, D), f32),
            jax.ShapeDtypeStruct((EH // 2, D), f32),
            jax.ShapeDtypeStruct((1, D), f32),
        ],
        **kwargs,
    )(*args)


def _scatter_body(ma_hbm, mb_hbm, dsta_hbm, dstb_hbm, zrow_hbm,
                  s_hbm,
                  dstbuf, dd0, dd1, dd2, dd3, dd4,
                  m0, m1, m2b, m3, m4, obuf,
                  l0, l1, l2, l3, l4, s0, s1, s2, s3, s4,
                  acc):
    c = lax.axis_index("c")
    s = lax.axis_index("s")
    mbufs = (m0, m1, m2b, m3, m4)
    dd = (dd0, dd1, dd2, dd3, dd4)
    lsems = (l0, l1, l2, l3, l4)
    ssems = (s0, s1, s2, s3, s4)

    def run(m2_hbm, dst_hbm):
        base = s * PERW
        loadbase = base - lax.rem(base, BLOCK_E)

        pltpu.sync_copy(zrow_hbm, obuf)
        pltpu.sync_copy(obuf, acc.at[pl.ds(s * RPT, RPT)])
        pltpu.sync_copy(dst_hbm.at[pl.ds(loadbase, DCOV)], dstbuf)
        plsc.subcore_barrier()

        def build_idx(ch, k):
            q0 = base + ch * SCH
            off = lax.rem(q0, BLOCK_E)
            r1 = (q0 - off) + lax.shift_right_logical(off, 1) - loadbase
            for g in range(SCH // 16):
                u = lax.iota(jnp.int32, 16) + 16 * g
                pos = (r1 + lax.bitwise_and(u, 1) * EPAIR
                       + lax.shift_right_logical(u, 1))
                dd[k][pl.ds(16 * g, 16)] = plsc.load_gather(dstbuf, [pos])

        def rows(ch):
            return m2_hbm.at[pl.ds(base + ch * SCH, SCH)]

        def l_start(ch, k):
            pltpu.async_copy(rows(ch), mbufs[k], lsems[k])

        for k in range(SSLOTS):
            l_start(k, k)

        def body(i, carry):
            ch0 = i * SSLOTS
            for k in range(SSLOTS):
                pltpu.make_async_copy(rows(ch0 + k), mbufs[k],
                                      lsems[k]).wait()
                build_idx(ch0 + k, k)
                pltpu.async_copy(mbufs[k], acc.at[dd[k]], ssems[k],
                                 add=True)
            for k in range(SSLOTS):
                @pl.when(i < SITER - 1)
                def _():
                    pltpu.make_async_copy(mbufs[k], acc.at[dd[k]],
                                          ssems[k]).wait()
                    l_start(ch0 + SSLOTS + k, k)
            return carry

        lax.fori_loop(0, SITER, body, 0)
        for k in range(SSLOTS):
            pltpu.make_async_copy(mbufs[k], acc.at[dd[k]], ssems[k]).wait()
        plsc.subcore_barrier()

        pltpu.sync_copy(acc.at[pl.ds(s * RPT, RPT)], obuf)
        pltpu.sync_copy(obuf, s_hbm.at[c].at[pl.ds(s * RPT, RPT)])

    @pl.when(c == 0)
    def _():
        run(ma_hbm, dsta_hbm)

    @pl.when(c == 1)
    def _():
        run(mb_hbm, dstb_hbm)


def _scatter_call(m2a, m2b, dsta, dstb, zrow):
    f32 = jnp.float32
    fn = pl.kernel(
        _scatter_body,
        out_type=[
            jax.ShapeDtypeStruct((NC, NPAD, H), f32),
        ],
        mesh=_MESH,
        compiler_params=_SC_PARAMS,
        scratch_types=(
            [pltpu.VMEM((DCOV,), jnp.int32)]
            + [pltpu.VMEM((SCH,), jnp.int32) for _ in range(SSLOTS)]
            + [pltpu.VMEM((SCH, H), f32) for _ in range(SSLOTS)]
            + [pltpu.VMEM((RPT, H), f32)]
            + [pltpu.SemaphoreType.DMA for _ in range(2 * SSLOTS)]
            + [pltpu.VMEM_SHARED((NPAD, H), f32)]
        ),
    )
    return fn(m2a, m2b, dsta, dstb, zrow)


def _node_body(nf, S, DG, u, uea, ueb, ga, Wv, bv, An, cun, W2n, b2n,
               Wau, Wae, Wav, b1a, W2a, b2a, ov, ou):
    v = _relu(_dot(nf[...], Wv[...]) + bv[...])
    ssum = S[0, :N, :] + S[1, :N, :]
    deg = DG[0, :N, 0:1] + DG[1, :N, 0:1]
    ve_c = ssum / jnp.maximum(deg, 1.0)
    h = _relu(_dot(v, An[...]) + ve_c + cun[...])
    v_new = _relu(_dot(h, W2n[...]) + b2n[...])
    ov[...] = v_new + nf[...]
    uv = jnp.sum(v_new, axis=0, keepdims=True) * (1.0 / N)
    ue = (uea[...] + ueb[...]) * (1.0 / E)
    ha = _relu(_dot(u[...], Wau[...]) + _dot(ue, Wae[...])
               + _dot(uv, Wav[...]) + b1a[...])
    u_new = _relu(_dot(ha, W2a[...]) + b2a[...])
    ou[...] = u_new + ga[...]


def _node_call(nf, S, DG, u, uea, ueb, ga, Wv, bv, An, cun, W2n, b2n,
               Wau, Wae, Wav, b1a, W2a, b2a):
    f32 = jnp.float32
    return pl.pallas_call(
        _node_body,
        out_shape=[
            jax.ShapeDtypeStruct((N, D), f32),
            jax.ShapeDtypeStruct((1, D), f32),
        ],
    )(nf, S, DG, u, uea, ueb, ga, Wv, bv, An, cun, W2n, b2n,
      Wau, Wae, Wav, b1a, W2a, b2a)


def kernel(edge_feat, node_feat, graph_attr, params, edge_index):
    f32 = jnp.float32
    (We, be), = params["edge_dense"]
    (Wv, bv), = params["node_dense"]
    (Wu, bu), = params["attr_dense"]
    (W1e, b1e), (W2e, b2e) = params["conv_edge"]
    (W1n, b1n), (W2n, b2n) = params["conv_node"]
    (W1a, b1a), (W2a, b2a) = params["conv_attr"]

    ABe = jnp.concatenate([W1e[0:64], W1e[64:128]], axis=1)
    Ce, De = W1e[128:192], W1e[192:256]
    An, Cn, Dn = W1n[0:64], W1n[64:192], W1n[192:256]
    Wau, Wae, Wav = W1a[0:64], W1a[64:192], W1a[192:320]
    row = lambda b: b.reshape(1, -1)

    src = edge_index[0]
    dst = edge_index[1]

    tcat, u, cue, cun = _prep_call(
        node_feat, graph_attr, Wv, row(bv), Wu, row(bu),
        ABe, De, row(b1e), Dn, row(b1n))
    t2 = tcat.reshape(2 * N, H)

    z16 = jnp.zeros((RPT, 16), f32)
    ones = jnp.ones((SCH, 16), f32)
    g2a, DG = _gather_a_call(t2, src[:EH], dst[:EH], dst, z16, ones)
    g2b, = _gather_b_call(t2, src[EH:], dst[EH:])

    ew = (We, row(be), Ce, W2e, row(b2e), Cn)
    oeA, m2pA, ueA = _edge_call(edge_feat, g2a.reshape(EH, D), cue, *ew,
                                half=0)
    oe, m2pB, ueB = _edge_call(edge_feat, g2b.reshape(EH, D), cue, *ew,
                               half=1, seed=oeA)

    zrow = jnp.zeros((RPT, H), f32)
    dsta = jnp.pad(dst[:EH], (0, DPAD))
    dstb = jnp.pad(dst[EH:], (0, DPAD))
    S, = _scatter_call(m2pA.reshape(EH, H), m2pB.reshape(EH, H),
                       dsta, dstb, zrow)

    out_v, out_u = _node_call(
        node_feat, S, DG, u, ueA, ueB, graph_attr, Wv, row(bv), An, cun,
        W2n, row(b2n), Wau, Wae, Wav, row(b1a), W2a, row(b2a))

    return (oe, out_v, out_u)


# speedup vs baseline: 6.4092x; 1.0335x over previous
import functools

import jax
import jax.numpy as jnp
from jax import lax
from jax.experimental import pallas as pl
from jax.experimental.pallas import tpu as pltpu
from jax.experimental.pallas import tpu_sc as plsc

N = 10000
E = 320000
EH = E // 2
D = 128
H = 64

NC = 2
NS = 16
NW = NC * NS

PERW_G = EH // NW
GCH = 40
GIDX = 2 * GCH
GNCH = PERW_G // GCH
GSLOTS = 5
GITER = GNCH // GSLOTS

PERW = 10000
SCH = 80
SNCH = PERW // SCH
SSLOTS = 5
SITER = SNCH // SSLOTS

BLOCK_E = 6400
GRID_H = EH // BLOCK_E
EPAIR = BLOCK_E // 2
DCOV = 19200
DPAD = 6400

NPAD = 10240
RPT = NPAD // NS

_relu = jax.nn.relu


def _dot(a, b):
    return jnp.dot(a, b, preferred_element_type=jnp.float32)


def _prep_body(nf, ga, Wv, bv, Wu, bu, ABe, De, b1e, Dn, b1n,
               t_o, u_o, cue_o, cun_o):
    v = _relu(_dot(nf[...], Wv[...]) + bv[...])
    t_o[...] = _dot(v, ABe[...])
    u = _relu(_dot(ga[...], Wu[...]) + bu[...])
    u_o[...] = u
    cue_o[...] = _dot(u, De[...]) + b1e[...]
    cun_o[...] = _dot(u, Dn[...]) + b1n[...]


def _prep_call(nf, ga, Wv, bv, Wu, bu, ABe, De, b1e, Dn, b1n):
    f32 = jnp.float32
    return pl.pallas_call(
        _prep_body,
        out_shape=[
            jax.ShapeDtypeStruct((N, D), f32),
            jax.ShapeDtypeStruct((1, H), f32),
            jax.ShapeDtypeStruct((1, H), f32),
            jax.ShapeDtypeStruct((1, H), f32),
        ],
    )(nf, ga, Wv, bv, Wu, bu, ABe, De, b1e, Dn, b1n)


def _gather_core(t2_hbm, g2_hbm, srcv, dstv, idxv, bufs, gsems, wsems,
                 base, obase):
    def bld(ch, carry):
        for g in range(GIDX // 16):
            p = lax.iota(jnp.int32, 16) + (ch * GIDX + 16 * g)
            e = lax.shift_right_logical(p, 1)
            par = lax.bitwise_and(p, 1)
            sv = plsc.load_gather(srcv, [e])
            dv = plsc.load_gather(dstv, [e])
            idxv[ch, pl.ds(16 * g, 16)] = jnp.where(par == 1,
                                                    2 * dv + 1, 2 * sv)
        return carry

    lax.fori_loop(0, GNCH, bld, 0)

    def g_start(ch, k):
        pltpu.async_copy(t2_hbm.at[idxv.at[ch]], bufs[k], gsems[k])

    def g_wait(ch, k):
        pltpu.make_async_copy(t2_hbm.at[idxv.at[ch]], bufs[k],
                              gsems[k]).wait()

    def w_rows(ch):
        return g2_hbm.at[pl.ds(obase + ch * GIDX, GIDX)]

    for k in range(GSLOTS):
        g_start(k, k)

    def body(i, carry):
        ch0 = i * GSLOTS
        for k in range(GSLOTS):
            g_wait(ch0 + k, k)
            pltpu.async_copy(bufs[k], w_rows(ch0 + k), wsems[k])
        for k in range(GSLOTS):
            @pl.when(i < GITER - 1)
            def _():
                pltpu.make_async_copy(bufs[k], w_rows(ch0 + k),
                                      wsems[k]).wait()
                g_start(ch0 + GSLOTS + k, k)
        return carry

    lax.fori_loop(0, GITER, body, 0)
    last = (GITER - 1) * GSLOTS
    for k in range(GSLOTS):
        pltpu.make_async_copy(bufs[k], w_rows(last + k), wsems[k]).wait()


def _gather_b_body(t2_hbm, src_hbm, dst_hbm, z16_hbm, ones_hbm,
                   g2_hbm, dg_hbm,
                   srcv, dstv, dstd, idxv, b0, b1, b2, b3, b4,
                   idxd, dbuf, ones, dsem,
                   g0, g1, g2s, g3, g4, w0, w1, w2, w3, w4, dacc):
    c = lax.axis_index("c")
    s = lax.axis_index("s")
    w = s * NC + c
    base = EH + w * PERW_G
    obase = 2 * w * PERW_G
    bufs = (b0, b1, b2, b3, b4)
    gsems = (g0, g1, g2s, g3, g4)
    wsems = (w0, w1, w2, w3, w4)

    pltpu.sync_copy(src_hbm.at[pl.ds(base, PERW_G)], srcv)
    pltpu.sync_copy(dst_hbm.at[pl.ds(base, PERW_G)], dstv)

    pltpu.sync_copy(z16_hbm, dbuf)
    pltpu.sync_copy(dbuf, dacc.at[pl.ds(s * RPT, RPT)])
    pltpu.sync_copy(ones_hbm, ones)
    pltpu.sync_copy(dst_hbm.at[pl.ds(w * PERW, PERW)], dstd)

    def bld_d(j, carry):
        for g in range(SCH // 16):
            idxd[j, pl.ds(16 * g, 16)] = dstd[pl.ds(j * SCH + 16 * g, 16)]
        return carry

    lax.fori_loop(0, SNCH, bld_d, 0)
    plsc.subcore_barrier()

    def deg_round(r, carry):
        for t in range(25):
            pltpu.async_copy(ones, dacc.at[idxd.at[r * 25 + t]], dsem,
                             add=True)
        for t in range(25):
            pltpu.make_async_copy(ones, dacc.at[idxd.at[r * 25 + t]],
                                  dsem).wait()
        return carry

    lax.fori_loop(0, SNCH // 25, deg_round, 0)

    _gather_core(t2_hbm, g2_hbm, srcv, dstv, idxv, bufs, gsems, wsems,
                 base, obase)

    plsc.subcore_barrier()
    pltpu.sync_copy(dacc.at[pl.ds(s * RPT, RPT)], dbuf)
    pltpu.sync_copy(dbuf, dg_hbm.at[c].at[pl.ds(s * RPT, RPT)])


def _gather_a_body(t2_hbm, src_hbm, dst_hbm,
                   g2_hbm,
                   srcv, dstv, idxv, b0, b1, b2, b3, b4,
                   g0, g1, g2s, g3, g4, w0, w1, w2, w3, w4):
    c = lax.axis_index("c")
    s = lax.axis_index("s")
    w = s * NC + c
    base = w * PERW_G
    obase = 2 * base
    bufs = (b0, b1, b2, b3, b4)
    gsems = (g0, g1, g2s, g3, g4)
    wsems = (w0, w1, w2, w3, w4)
    pltpu.sync_copy(src_hbm.at[pl.ds(base, PERW_G)], srcv)
    pltpu.sync_copy(dst_hbm.at[pl.ds(base, PERW_G)], dstv)
    _gather_core(t2_hbm, g2_hbm, srcv, dstv, idxv, bufs, gsems, wsems,
                 base, obase)


_SC_PARAMS = pltpu.CompilerParams(use_tc_tiling_on_sc=False,
                                  needs_layout_passes=False)
_MESH = plsc.VectorSubcoreMesh(core_axis_name="c", subcore_axis_name="s")


def _gather_b_call(t2, src_full, dst_full, z16, ones):
    f32 = jnp.float32
    fn = pl.kernel(
        _gather_b_body,
        out_type=[
            jax.ShapeDtypeStruct((2 * EH, H), f32),
            jax.ShapeDtypeStruct((NC, NPAD, 16), f32),
        ],
        mesh=_MESH,
        compiler_params=_SC_PARAMS,
        scratch_types=(
            [pltpu.VMEM((PERW_G,), jnp.int32),
             pltpu.VMEM((PERW_G,), jnp.int32),
             pltpu.VMEM((PERW,), jnp.int32),
             pltpu.VMEM((GNCH, GIDX), jnp.int32)]
            + [pltpu.VMEM((GIDX, H), f32) for _ in range(GSLOTS)]
            + [pltpu.VMEM((SNCH, SCH), jnp.int32),
               pltpu.VMEM((RPT, 16), f32),
               pltpu.VMEM((SCH, 16), f32),
               pltpu.SemaphoreType.DMA]
            + [pltpu.SemaphoreType.DMA for _ in range(2 * GSLOTS)]
            + [pltpu.VMEM_SHARED((NPAD, 16), f32)]
        ),
    )
    return fn(t2, src_full, dst_full, z16, ones)


def _gather_a_call(t2, src_full, dst_full):
    f32 = jnp.float32
    fn = pl.kernel(
        _gather_a_body,
        out_type=[jax.ShapeDtypeStruct((2 * EH, H), f32)],
        mesh=_MESH,
        compiler_params=_SC_PARAMS,
        scratch_types=(
            [pltpu.VMEM((PERW_G,), jnp.int32),
             pltpu.VMEM((PERW_G,), jnp.int32),
             pltpu.VMEM((GNCH, GIDX), jnp.int32)]
            + [pltpu.VMEM((GIDX, H), f32) for _ in range(GSLOTS)]
            + [pltpu.SemaphoreType.DMA for _ in range(2 * GSLOTS)]
        ),
    )
    return fn(t2, src_full, dst_full)


def _edge_body_a(ef, g, cue, We, be, Ce, W2e, b2e, Cn, oe, m2, ue):
    _edge_compute(ef, g, cue, We, be, Ce, W2e, b2e, Cn, oe, m2, ue)


def _edge_body_b(seed, ef, g, cue, We, be, Ce, W2e, b2e, Cn, oe, m2, ue):
    _edge_compute(ef, g, cue, We, be, Ce, W2e, b2e, Cn, oe, m2, ue)


def _edge_compute(ef, g, cue, We, be, Ce, W2e, b2e, Cn, oe, m2, ue):
    i = pl.program_id(0)
    e1 = _relu(_dot(ef[...], We[...]) + be[...])
    gg = g[...]
    h1 = _relu(_dot(e1, Ce[...]) + gg[:, :H] + gg[:, H:] + cue[...])
    mij = _relu(_dot(h1, W2e[...]) + b2e[...])
    oe[...] = mij + ef[...]
    nb = mij.shape[0] // 2
    m2[...] = jnp.concatenate(
        [_dot(mij[:nb], Cn[...]), _dot(mij[nb:], Cn[...])], axis=1)

    @pl.when(i == 0)
    def _():
        ue[...] = jnp.zeros_like(ue)

    ue[...] += jnp.sum(mij, axis=0, keepdims=True)


def _edge_call(ef, g, cue, We, be, Ce, W2e, b2e, Cn, half, seed=None):
    f32 = jnp.float32
    full = lambda i: (0, 0)
    off = half * GRID_H
    in_specs = [
        pl.BlockSpec((BLOCK_E, D), lambda i: (i + off, 0)),
        pl.BlockSpec((BLOCK_E, D), lambda i: (i, 0)),
        pl.BlockSpec((1, H), full),
        pl.BlockSpec((D, H), full),
        pl.BlockSpec((1, H), full),
        pl.BlockSpec((H, H), full),
        pl.BlockSpec((H, D), full),
        pl.BlockSpec((1, D), full),
        pl.BlockSpec((D, H), full),
    ]
    args = [ef, g, cue, We, be, Ce, W2e, b2e, Cn]
    kwargs = {}
    if seed is None:
        body = _edge_body_a
    else:
        body = _edge_body_b
        in_specs = [pl.BlockSpec((8, D), full)] + in_specs
        args = [seed] + args
        kwargs["input_output_aliases"] = {0: 0}
    return pl.pallas_call(
        body,
        grid=(GRID_H,),
        in_specs=in_specs,
        out_specs=[
            pl.BlockSpec((BLOCK_E, D), lambda i: (i + off, 0)),
            pl.BlockSpec((BLOCK_E // 2, D), lambda i: (i, 0)),
            pl.BlockSpec((1, D), full),
        ],
        out_shape=[
            jax.ShapeDtypeStruct((E, D), f32),
            jax.ShapeDtypeStruct((EH // 2, D), f32),
            jax.ShapeDtypeStruct((1, D), f32),
        ],
        **kwargs,
    )(*args)


def _scatter_body(ma_hbm, mb_hbm, dst_hbm, zrow_hbm,
                  s_hbm,
                  dstbuf, dd0, dd1, dd2, dd3, dd4,
                  m0, m1, m2b, m3, m4, obuf,
                  l0, l1, l2, l3, l4, s0, s1, s2, s3, s4,
                  acc):
    c = lax.axis_index("c")
    s = lax.axis_index("s")
    mbufs = (m0, m1, m2b, m3, m4)
    dd = (dd0, dd1, dd2, dd3, dd4)
    lsems = (l0, l1, l2, l3, l4)
    ssems = (s0, s1, s2, s3, s4)

    def run(m2_hbm, half):
        base_d = half * EH + s * PERW
        base = s * PERW
        loadbase = base_d - lax.rem(base_d, BLOCK_E)

        pltpu.sync_copy(zrow_hbm, obuf)
        pltpu.sync_copy(obuf, acc.at[pl.ds(s * RPT, RPT)])
        pltpu.sync_copy(dst_hbm.at[pl.ds(loadbase, DCOV)], dstbuf)
        plsc.subcore_barrier()

        def build_idx(ch, k):
            q0 = base_d + ch * SCH
            off = lax.rem(q0, BLOCK_E)
            r1 = (q0 - off) + lax.shift_right_logical(off, 1) - loadbase
            for g in range(SCH // 16):
                u = lax.iota(jnp.int32, 16) + 16 * g
                pos = (r1 + lax.bitwise_and(u, 1) * EPAIR
                       + lax.shift_right_logical(u, 1))
                dd[k][pl.ds(16 * g, 16)] = plsc.load_gather(dstbuf, [pos])

        def rows(ch):
            return m2_hbm.at[pl.ds(base + ch * SCH, SCH)]

        def l_start(ch, k):
            pltpu.async_copy(rows(ch), mbufs[k], lsems[k])

        for k in range(SSLOTS):
            l_start(k, k)

        def body(i, carry):
            ch0 = i * SSLOTS
            for k in range(SSLOTS):
                pltpu.make_async_copy(rows(ch0 + k), mbufs[k],
                                      lsems[k]).wait()
                build_idx(ch0 + k, k)
                pltpu.async_copy(mbufs[k], acc.at[dd[k]], ssems[k],
                                 add=True)
            for k in range(SSLOTS):
                @pl.when(i < SITER - 1)
                def _():
                    pltpu.make_async_copy(mbufs[k], acc.at[dd[k]],
                                          ssems[k]).wait()
                    l_start(ch0 + SSLOTS + k, k)
            return carry

        lax.fori_loop(0, SITER, body, 0)
        for k in range(SSLOTS):
            pltpu.make_async_copy(mbufs[k], acc.at[dd[k]], ssems[k]).wait()
        plsc.subcore_barrier()

        pltpu.sync_copy(acc.at[pl.ds(s * RPT, RPT)], obuf)
        pltpu.sync_copy(obuf, s_hbm.at[c].at[pl.ds(s * RPT, RPT)])

    @pl.when(c == 0)
    def _():
        run(ma_hbm, 0)

    @pl.when(c == 1)
    def _():
        run(mb_hbm, 1)


def _scatter_call(m2a, m2b, dstp, zrow):
    f32 = jnp.float32
    fn = pl.kernel(
        _scatter_body,
        out_type=[
            jax.ShapeDtypeStruct((NC, NPAD, H), f32),
        ],
        mesh=_MESH,
        compiler_params=_SC_PARAMS,
        scratch_types=(
            [pltpu.VMEM((DCOV,), jnp.int32)]
            + [pltpu.VMEM((SCH,), jnp.int32) for _ in range(SSLOTS)]
            + [pltpu.VMEM((SCH, H), f32) for _ in range(SSLOTS)]
            + [pltpu.VMEM((RPT, H), f32)]
            + [pltpu.SemaphoreType.DMA for _ in range(2 * SSLOTS)]
            + [pltpu.VMEM_SHARED((NPAD, H), f32)]
        ),
    )
    return fn(m2a, m2b, dstp, zrow)


def _node_body(nf, S, DG, u, uea, ueb, ga, Wv, bv, An, cun, W2n, b2n,
               Wau, Wae, Wav, b1a, W2a, b2a, ov, ou):
    v = _relu(_dot(nf[...], Wv[...]) + bv[...])
    ssum = S[0, :N, :] + S[1, :N, :]
    deg = DG[0, :N, 0:1] + DG[1, :N, 0:1]
    ve_c = ssum / jnp.maximum(deg, 1.0)
    h = _relu(_dot(v, An[...]) + ve_c + cun[...])
    v_new = _relu(_dot(h, W2n[...]) + b2n[...])
    ov[...] = v_new + nf[...]
    uv = jnp.sum(v_new, axis=0, keepdims=True) * (1.0 / N)
    ue = (uea[...] + ueb[...]) * (1.0 / E)
    ha = _relu(_dot(u[...], Wau[...]) + _dot(ue, Wae[...])
               + _dot(uv, Wav[...]) + b1a[...])
    u_new = _relu(_dot(ha, W2a[...]) + b2a[...])
    ou[...] = u_new + ga[...]


def _node_call(nf, S, DG, u, uea, ueb, ga, Wv, bv, An, cun, W2n, b2n,
               Wau, Wae, Wav, b1a, W2a, b2a):
    f32 = jnp.float32
    return pl.pallas_call(
        _node_body,
        out_shape=[
            jax.ShapeDtypeStruct((N, D), f32),
            jax.ShapeDtypeStruct((1, D), f32),
        ],
    )(nf, S, DG, u, uea, ueb, ga, Wv, bv, An, cun, W2n, b2n,
      Wau, Wae, Wav, b1a, W2a, b2a)


def kernel(edge_feat, node_feat, graph_attr, params, edge_index):
    f32 = jnp.float32
    (We, be), = params["edge_dense"]
    (Wv, bv), = params["node_dense"]
    (Wu, bu), = params["attr_dense"]
    (W1e, b1e), (W2e, b2e) = params["conv_edge"]
    (W1n, b1n), (W2n, b2n) = params["conv_node"]
    (W1a, b1a), (W2a, b2a) = params["conv_attr"]

    ABe = jnp.concatenate([W1e[0:64], W1e[64:128]], axis=1)
    Ce, De = W1e[128:192], W1e[192:256]
    An, Cn, Dn = W1n[0:64], W1n[64:192], W1n[192:256]
    Wau, Wae, Wav = W1a[0:64], W1a[64:192], W1a[192:320]
    row = lambda b: b.reshape(1, -1)

    src = edge_index[0]
    dst = edge_index[1]

    tcat, u, cue, cun = _prep_call(
        node_feat, graph_attr, Wv, row(bv), Wu, row(bu),
        ABe, De, row(b1e), Dn, row(b1n))
    t2 = tcat.reshape(2 * N, H)

    z16 = jnp.zeros((RPT, 16), f32)
    ones = jnp.ones((SCH, 16), f32)
    g2a, = _gather_a_call(t2, src, dst)
    g2b, DG = _gather_b_call(t2, src, dst, z16, ones)

    ew = (We, row(be), Ce, W2e, row(b2e), Cn)
    oeA, m2pA, ueA = _edge_call(edge_feat, g2a.reshape(EH, D), cue, *ew,
                                half=0)
    oe, m2pB, ueB = _edge_call(edge_feat, g2b.reshape(EH, D), cue, *ew,
                               half=1, seed=oeA)

    zrow = jnp.zeros((RPT, H), f32)
    dstp = jnp.pad(dst, (0, DPAD))
    S, = _scatter_call(m2pA.reshape(EH, H), m2pB.reshape(EH, H),
                       dstp, zrow)

    out_v, out_u = _node_call(
        node_feat, S, DG, u, ueA, ueB, graph_attr, Wv, row(bv), An, cun,
        W2n, row(b2n), Wau, Wae, Wav, row(b1a), W2a, row(b2a))

    return (oe, out_v, out_u)
